# baseline (speedup 1.0000x reference)
import functools
import math

import jax
import jax.numpy as jnp
from jax import lax
from jax.experimental import pallas as pl
from jax.experimental.pallas import tpu as pltpu
from jax.experimental.pallas import tpu_sc as plsc

DMODEL = 64
SCALE = math.sqrt(DMODEL)
NROWS, NCOLS = 4096, 200
B = NROWS * NCOLS
NC, NS = 2, 16
NW = NC * NS
BPW = B // NW
CHUNK = 128
NCHUNK = BPW // CHUNK
NBUF = 4
LANES = 16


def _scale_rows(buf):
    @pl.loop(0, CHUNK)
    def _(i):
        for k in range(DMODEL // LANES):
            sl = pl.ds(k * LANES, LANES)
            buf[i, sl] = buf[i, sl] * SCALE


def _emb_body(x_hbm, table_hbm, out_hbm, idx_all, bufs, sems):
    wid = lax.axis_index("s") * NC + lax.axis_index("c")
    crow = wid * NCHUNK
    rbase = wid * BPW

    pltpu.sync_copy(x_hbm.at[pl.ds(crow, NCHUNK)], idx_all)

    def gather_start(j, b):
        pltpu.async_copy(table_hbm.at[idx_all.at[j]], bufs[b], sems[b])

    def gather_wait(j, b):
        pltpu.make_async_copy(table_hbm.at[idx_all.at[j]], bufs[b], sems[b]).wait()

    def finish_chunk(j, b):
        _scale_rows(bufs[b])
        pltpu.sync_copy(bufs[b], out_hbm.at[pl.ds(rbase + j * CHUNK, CHUNK)])

    for b in range(NBUF):
        gather_start(b, b)

    @pl.loop(0, NCHUNK - NBUF, step=NBUF)
    def _(j0):
        for b in range(NBUF):
            j = j0 + b
            gather_wait(j, b)
            finish_chunk(j, b)
            gather_start(j + NBUF, b)

    for b in range(NBUF):
        j = NCHUNK - NBUF + b
        gather_wait(j, b)
        finish_chunk(j, b)


@functools.partial(
    pl.kernel,
    out_type=jax.ShapeDtypeStruct((B, DMODEL), jnp.float32),
    mesh=plsc.VectorSubcoreMesh(core_axis_name="c", subcore_axis_name="s"),
    scratch_types=dict(
        idx_all=pltpu.VMEM((NCHUNK, CHUNK), jnp.int32),
        bufs=[pltpu.VMEM((CHUNK, DMODEL), jnp.float32) for _ in range(NBUF)],
        sems=[pltpu.SemaphoreType.DMA for _ in range(NBUF)],
    ),
    compiler_params=pltpu.CompilerParams(use_tc_tiling_on_sc=False),
)
def _emb(x_hbm, table_hbm, out_hbm, idx_all, bufs, sems):
    _emb_body(x_hbm, table_hbm, out_hbm, idx_all, bufs, sems)


def kernel(x, table):
    xf = x.astype(jnp.int32).reshape(B // CHUNK, CHUNK)
    out = _emb(xf, table)
    return out.reshape(NROWS, NCOLS, DMODEL)
